# baseline (speedup 1.0000x reference)
import functools

import jax
import jax.numpy as jnp
from jax import lax
from jax.experimental import pallas as pl
from jax.experimental.pallas import tpu as pltpu
from jax.experimental.pallas import tpu_sc as plsc

N_ROWS = 100000
NUM_CLASSES = 100
EMB_DIM = 128
NSF = 9
OUT_DIM = 256
XS_BASE = 120

S_SC = 40960
N_TC = N_ROWS - S_SC
BLOCK_R = 4920
NW = 32
B_PER_W = S_SC // NW
CHUNK = 128
N_CHUNKS = B_PER_W // CHUNK


def _tc_body(x_ref, emb_ref, w1t_ref, w2t_ref, b_ref, out_ref, m_ref):
    @pl.when(pl.program_id(0) == 0)
    def _init():
        m = jnp.dot(emb_ref[...], w1t_ref[...], preferred_element_type=jnp.float32)
        row = jax.lax.broadcasted_iota(jnp.int32, (EMB_DIM, OUT_DIM), 0)
        m_ref[...] = m + jnp.where(row <= NUM_CLASSES, b_ref[...], 0.0)
        m_ref[XS_BASE:XS_BASE + NSF - 1, :] = w2t_ref[...]

    xb = x_ref[...]
    laneb = jax.lax.broadcasted_iota(jnp.int32, (1, EMB_DIM), 1).astype(jnp.bfloat16)
    onehot = jnp.where(laneb == xb[:, 0:1].astype(jnp.bfloat16),
                       jnp.bfloat16(1), jnp.bfloat16(0))
    xs = xb[:, 1:NSF].astype(jnp.bfloat16)
    shifted = jnp.concatenate(
        [jnp.zeros((BLOCK_R, XS_BASE), jnp.bfloat16), xs], axis=1)
    a = onehot + shifted
    out_ref[...] = jnp.dot(a, m_ref[...].astype(jnp.bfloat16),
                           preferred_element_type=jnp.float32)


def _tc_part(x_tail, emb_pad, w1t, w2t, b2d):
    return pl.pallas_call(
        _tc_body,
        grid=(N_TC // BLOCK_R,),
        in_specs=[
            pl.BlockSpec((BLOCK_R, NSF), lambda i: (i, 0)),
            pl.BlockSpec((EMB_DIM, EMB_DIM), lambda i: (0, 0)),
            pl.BlockSpec((EMB_DIM, OUT_DIM), lambda i: (0, 0)),
            pl.BlockSpec((NSF - 1, OUT_DIM), lambda i: (0, 0)),
            pl.BlockSpec((1, OUT_DIM), lambda i: (0, 0)),
        ],
        out_specs=pl.BlockSpec((BLOCK_R, OUT_DIM), lambda i: (i, 0)),
        out_shape=jax.ShapeDtypeStruct((N_TC, OUT_DIM), jnp.float32),
        scratch_shapes=[pltpu.VMEM((EMB_DIM, OUT_DIM), jnp.float32)],
    )(x_tail, emb_pad, w1t, w2t, b2d)


def _sc_gather(table, idx):
    mesh = plsc.VectorSubcoreMesh(core_axis_name="c", subcore_axis_name="s")

    @functools.partial(
        pl.kernel, mesh=mesh,
        out_type=jax.ShapeDtypeStruct((S_SC, OUT_DIM), jnp.float32),
        scratch_types=[
            pltpu.VMEM((B_PER_W,), jnp.int32),
            pltpu.VMEM((CHUNK, OUT_DIM), jnp.float32),
            pltpu.VMEM((CHUNK, OUT_DIM), jnp.float32),
            pltpu.SemaphoreType.DMA,
            pltpu.SemaphoreType.DMA,
            pltpu.SemaphoreType.DMA,
            pltpu.SemaphoreType.DMA,
        ],
    )
    def k(table_hbm, idx_hbm, out_hbm, idx_v, rows0, rows1, sg0, sg1, ss0, ss1):
        wid = lax.axis_index("s") * 2 + lax.axis_index("c")
        base = wid * B_PER_W
        pltpu.sync_copy(idx_hbm.at[pl.ds(base, B_PER_W)], idx_v)
        bufs = (rows0, rows1)
        gsems = (sg0, sg1)
        ssems = (ss0, ss1)
        gd = [None] * N_CHUNKS
        sd = [None] * N_CHUNKS
        gd[0] = pltpu.async_copy(
            table_hbm.at[idx_v.at[pl.ds(0, CHUNK)]], bufs[0], gsems[0])
        for c in range(N_CHUNKS):
            nxt = c + 1
            if nxt < N_CHUNKS:
                if sd[nxt - 2] is not None:
                    sd[nxt - 2].wait()
                gd[nxt] = pltpu.async_copy(
                    table_hbm.at[idx_v.at[pl.ds(nxt * CHUNK, CHUNK)]],
                    bufs[nxt % 2], gsems[nxt % 2])
            gd[c].wait()
            sd[c] = pltpu.async_copy(
                bufs[c % 2], out_hbm.at[pl.ds(base + c * CHUNK, CHUNK)],
                ssems[c % 2])
        sd[N_CHUNKS - 2].wait()
        sd[N_CHUNKS - 1].wait()

    return k(table, idx)


def kernel(x, emb_table, W, b):
    if x.ndim == 1:
        x = x[:, None]
    emb_pad = jnp.pad(emb_table, ((0, EMB_DIM - (NUM_CLASSES + 1)), (0, 0)))
    w1t = W[:, :EMB_DIM].T
    w2t = W[:, EMB_DIM:].T
    b2d = b.reshape(1, OUT_DIM)
    table = emb_pad @ w1t + b2d
    idx = x[:S_SC, 0].astype(jnp.int32)
    sc_out = _sc_gather(table, idx)
    tc_out = _tc_part(x[S_SC:], emb_pad, w1t, w2t, b2d)
    return tc_out, sc_out


# speedup vs baseline: 1.7018x; 1.7018x over previous
import jax
import jax.numpy as jnp
from jax.experimental import pallas as pl
from jax.experimental.pallas import tpu as pltpu

N_ROWS = 100000
NUM_CLASSES = 100
EMB_DIM = 128
NSF = 9
OUT_DIM = 256
BLOCK_R = 10000
XS_BASE = 120


def _body(x_ref, emb_ref, w1t_ref, w2t_ref, b_ref, out_ref, m_ref):
    @pl.when(pl.program_id(0) == 0)
    def _init():
        m = jnp.dot(emb_ref[...], w1t_ref[...], preferred_element_type=jnp.float32)
        row = jax.lax.broadcasted_iota(jnp.int32, (EMB_DIM, OUT_DIM), 0)
        m_ref[...] = m + jnp.where(row <= NUM_CLASSES, b_ref[...], 0.0)
        m_ref[XS_BASE:XS_BASE + NSF - 1, :] = w2t_ref[...]

    xb = x_ref[...]
    laneb = jax.lax.broadcasted_iota(jnp.int32, (1, EMB_DIM), 1).astype(jnp.bfloat16)
    onehot = jnp.where(laneb == xb[:, 0:1].astype(jnp.bfloat16),
                       jnp.bfloat16(1), jnp.bfloat16(0))
    xs = xb[:, 1:NSF].astype(jnp.bfloat16)
    shifted = jnp.concatenate(
        [jnp.zeros((BLOCK_R, XS_BASE), jnp.bfloat16), xs], axis=1)
    a = onehot + shifted
    out_ref[...] = jnp.dot(a, m_ref[...].astype(jnp.bfloat16),
                           preferred_element_type=jnp.float32)


def kernel(x, emb_table, W, b):
    if x.ndim == 1:
        x = x[:, None]
    emb_pad = jnp.pad(emb_table, ((0, EMB_DIM - (NUM_CLASSES + 1)), (0, 0)))
    w1t = W[:, :EMB_DIM].T
    w2t = W[:, EMB_DIM:].T
    b2d = b.reshape(1, OUT_DIM)
    grid = (N_ROWS // BLOCK_R,)
    return pl.pallas_call(
        _body,
        grid=grid,
        in_specs=[
            pl.BlockSpec((BLOCK_R, NSF), lambda i: (i, 0)),
            pl.BlockSpec((EMB_DIM, EMB_DIM), lambda i: (0, 0)),
            pl.BlockSpec((EMB_DIM, OUT_DIM), lambda i: (0, 0)),
            pl.BlockSpec((NSF - 1, OUT_DIM), lambda i: (0, 0)),
            pl.BlockSpec((1, OUT_DIM), lambda i: (0, 0)),
        ],
        out_specs=pl.BlockSpec((BLOCK_R, OUT_DIM), lambda i: (i, 0)),
        out_shape=jax.ShapeDtypeStruct((N_ROWS, OUT_DIM), jnp.float32),
        scratch_shapes=[pltpu.VMEM((EMB_DIM, OUT_DIM), jnp.float32)],
    )(x, emb_pad, w1t, w2t, b2d)
